# Initial kernel scaffold; baseline (speedup 1.0000x reference)
import jax
import jax.numpy as jnp
from jax.experimental import pallas as pl


def kernel(node_features, edge_features, from_idx, to_idx, query_sizes, corpus_sizes, W_ne, b_ne, W_ee, b_ee, W_msg, b_msg, W_upd, b_upd, W_a1, b_a1, W_a2, b_a2):
    raise NotImplementedError("write your pallas kernel here")


# speedup vs baseline: 4.6643x; 4.6643x over previous
import functools

import jax
import jax.numpy as jnp
from jax import lax
from jax.experimental import pallas as pl
from jax.experimental.pallas import tpu as pltpu
from jax.experimental.pallas import tpu_sc as plsc

N_NODES = 24576
N_EDGES = 393216
D_NODE = 128
D_EDGE_IN = 4
D_EDGE = 16
D_ATT = 64
NUM_PAIRS = 256
NODES_PER_GRAPH = 48
MAX_SET = 64
STEPS = 3
TEMP = 0.1

NC = 2
NS = 16
HALF = D_NODE // NC
ROWS_PER_TILE = N_NODES // NS
CHUNK = 128
EDGE_ROWS = N_EDGES // CHUNK
ROWS_PER_TILE_SPMM = EDGE_ROWS // NS
ROWS_PER_WORKER_P0 = EDGE_ROWS // (NC * NS)

BLK = 512
N_BLKS = N_NODES // BLK


def _prep_body(wmsg_ref, wupd_ref, m8_ref, bm8_ref, wcat_ref, wtp_ref, we8_ref):
    wub = wupd_ref[D_NODE:, :]
    wcat_ref[:D_NODE, :] = wupd_ref[:D_NODE, :]
    wcat_ref[D_NODE:, :] = jnp.dot(wmsg_ref[:D_NODE, :], wub,
                                   preferred_element_type=jnp.float32)
    wtp_ref[...] = jnp.dot(wmsg_ref[D_NODE:2 * D_NODE, :], wub,
                           preferred_element_type=jnp.float32)
    tmp = jnp.dot(m8_ref[...], wmsg_ref[2 * D_NODE:, :],
                  preferred_element_type=jnp.float32) + bm8_ref[...]
    we8_ref[...] = jnp.dot(tmp, wub, preferred_element_type=jnp.float32)


def _prep_weights(W_msg, W_upd, M8, bm8):
    return pl.pallas_call(
        _prep_body,
        out_shape=(
            jax.ShapeDtypeStruct((2 * D_NODE, D_NODE), jnp.float32),
            jax.ShapeDtypeStruct((D_NODE, D_NODE), jnp.float32),
            jax.ShapeDtypeStruct((8, D_NODE), jnp.float32),
        ),
    )(W_msg, W_upd, M8, bm8)


def _enc_body(x_ref, w_ref, b_ref, out_ref):
    y = jnp.dot(x_ref[...], w_ref[...], preferred_element_type=jnp.float32)
    y = y + b_ref[...]
    out_ref[0] = y[:, :HALF]
    out_ref[1] = y[:, HALF:]


def _encoder(x, W_ne, b_ne):
    return pl.pallas_call(
        _enc_body,
        grid=(N_BLKS,),
        in_specs=[
            pl.BlockSpec((BLK, D_NODE), lambda i: (i, 0)),
            pl.BlockSpec((D_NODE, D_NODE), lambda i: (0, 0)),
            pl.BlockSpec((1, D_NODE), lambda i: (0, 0)),
        ],
        out_specs=pl.BlockSpec((NC, BLK, HALF), lambda i: (0, i, 0)),
        out_shape=jax.ShapeDtypeStruct((NC, N_NODES, HALF), jnp.float32),
    )(x, W_ne, b_ne)


def _pass0_body(e2d, to2d, zrows, zacc, out, tidx, ev, rows8, acc8):
    c = lax.axis_index("c")
    s = lax.axis_index("s")
    pltpu.sync_copy(zacc, acc8.at[pl.ds(s * ROWS_PER_TILE, ROWS_PER_TILE)])
    pltpu.sync_copy(zrows, rows8)
    lane = jnp.arange(16, dtype=jnp.int32)
    ones = jnp.ones((16,), jnp.float32)
    col4 = jnp.full((16,), 4, jnp.int32)
    for k in range(8):
        plsc.store_scatter(rows8, [k * 16 + lane, col4], ones)
    plsc.subcore_barrier()

    base = c * (NS * ROWS_PER_WORKER_P0) + s * ROWS_PER_WORKER_P0
    rowpat = lane >> 2
    colpat = lane & 3

    def chunk(r, carry):
        row = base + r
        pltpu.sync_copy(to2d.at[row], tidx)
        pltpu.sync_copy(e2d.at[pl.ds(row * 4, 4)], ev)
        for j in range(4):
            for k in range(8):
                v = ev[j, pl.ds(k * 16, 16)]
                base_t = j * 32 + k * 4
                plsc.store_scatter(rows8, [base_t + rowpat, colpat], v)
        pltpu.sync_copy(rows8, acc8.at[tidx], add=True)
        return carry

    lax.fori_loop(0, ROWS_PER_WORKER_P0, chunk, 0)
    plsc.subcore_barrier()
    sl = pl.ds(s * ROWS_PER_TILE, ROWS_PER_TILE)
    pltpu.sync_copy(acc8.at[sl], out.at[c].at[sl])


def _pass0(edge_features, to2d):
    e2d = edge_features.reshape(N_EDGES * D_EDGE_IN // CHUNK, CHUNK)
    mesh = plsc.VectorSubcoreMesh(core_axis_name="c", subcore_axis_name="s",
                                  num_cores=NC, num_subcores=NS)
    zrows = jnp.zeros((CHUNK, 8), jnp.float32)
    zacc = jnp.zeros((ROWS_PER_TILE, 8), jnp.float32)
    f = pl.kernel(
        _pass0_body,
        out_type=jax.ShapeDtypeStruct((NC, N_NODES, 8), jnp.float32),
        mesh=mesh,
        compiler_params=pltpu.CompilerParams(use_tc_tiling_on_sc=False,
                                             needs_layout_passes=False),
        scratch_types=[
            pltpu.VMEM((CHUNK,), jnp.int32),
            pltpu.VMEM((4, CHUNK), jnp.float32),
            pltpu.VMEM((CHUNK, 8), jnp.float32),
            pltpu.VMEM_SHARED((N_NODES, 8), jnp.float32),
        ],
    )
    return f(e2d, to2d, zrows, zacc)


def _spmm_body(h3, from2d, to2d, zeros, out, fidx, tidx, rows, acc, sem):
    c = lax.axis_index("c")
    s = lax.axis_index("s")
    pltpu.sync_copy(zeros, acc.at[pl.ds(s * ROWS_PER_TILE, ROWS_PER_TILE)])
    plsc.subcore_barrier()

    def chunk(r, carry):
        row = s * ROWS_PER_TILE_SPMM + r
        pltpu.sync_copy(from2d.at[row], fidx)
        pltpu.sync_copy(to2d.at[row], tidx)
        pltpu.async_copy(h3.at[c].at[fidx], rows, sem).wait()
        pltpu.sync_copy(rows, acc.at[tidx], add=True)
        return carry

    lax.fori_loop(0, ROWS_PER_TILE_SPMM, chunk, 0)
    plsc.subcore_barrier()
    sl = pl.ds(s * ROWS_PER_TILE, ROWS_PER_TILE)
    pltpu.sync_copy(acc.at[sl], out.at[c].at[sl])


def _spmm(h3, from2d, to2d, zeros):
    mesh = plsc.VectorSubcoreMesh(core_axis_name="c", subcore_axis_name="s",
                                  num_cores=NC, num_subcores=NS)
    f = pl.kernel(
        _spmm_body,
        out_type=jax.ShapeDtypeStruct((NC, N_NODES, HALF), jnp.float32),
        mesh=mesh,
        compiler_params=pltpu.CompilerParams(use_tc_tiling_on_sc=False),
        scratch_types=[
            pltpu.VMEM((CHUNK,), jnp.int32),
            pltpu.VMEM((CHUNK,), jnp.int32),
            pltpu.VMEM((CHUNK, HALF), jnp.float32),
            pltpu.VMEM_SHARED((N_NODES, HALF), jnp.float32),
            pltpu.SemaphoreType.DMA,
        ],
    )
    return f(h3, from2d, to2d, zeros)


def _step_body(h_ref, a_ref, e_ref, wcat_ref, wtp_ref, we8_ref, b_ref, out_ref):
    h = jnp.concatenate([h_ref[0], h_ref[1]], axis=1)
    a = jnp.concatenate([a_ref[0], a_ref[1]], axis=1)
    eraw = e_ref[0] + e_ref[1]
    cnt = eraw[:, 4:5]
    x = jnp.concatenate([h, a], axis=1)
    y = jnp.dot(x, wcat_ref[...], preferred_element_type=jnp.float32)
    y = y + jnp.dot(cnt * h, wtp_ref[...], preferred_element_type=jnp.float32)
    y = y + jnp.dot(eraw, we8_ref[...], preferred_element_type=jnp.float32)
    y = y + b_ref[...]
    out_ref[0] = y[:, :HALF]
    out_ref[1] = y[:, HALF:]


def _step(h3, a3, e2, Wcat, Wtp, We8, b_upd):
    return pl.pallas_call(
        _step_body,
        grid=(N_BLKS,),
        in_specs=[
            pl.BlockSpec((NC, BLK, HALF), lambda i: (0, i, 0)),
            pl.BlockSpec((NC, BLK, HALF), lambda i: (0, i, 0)),
            pl.BlockSpec((NC, BLK, 8), lambda i: (0, i, 0)),
            pl.BlockSpec((2 * D_NODE, D_NODE), lambda i: (0, 0)),
            pl.BlockSpec((D_NODE, D_NODE), lambda i: (0, 0)),
            pl.BlockSpec((8, D_NODE), lambda i: (0, 0)),
            pl.BlockSpec((1, D_NODE), lambda i: (0, 0)),
        ],
        out_specs=pl.BlockSpec((NC, BLK, HALF), lambda i: (0, i, 0)),
        out_shape=jax.ShapeDtypeStruct((NC, N_NODES, HALF), jnp.float32),
    )(h3, a3, e2, Wcat, Wtp, We8, b_upd)


def _attn_body(qlo_ref, qhi_ref, clo_ref, chi_ref, qs_ref, cs_ref,
               wa1_ref, ba1_ref, wa2_ref, ba2_ref, out_ref):
    p = pl.program_id(0)
    zpad = jnp.zeros((MAX_SET - NODES_PER_GRAPH, D_NODE), jnp.float32)
    q = jnp.concatenate(
        [jnp.concatenate([qlo_ref[0], qhi_ref[0]], axis=1), zpad], axis=0)
    c = jnp.concatenate(
        [jnp.concatenate([clo_ref[0], chi_ref[0]], axis=1), zpad], axis=0)
    qs = qs_ref[p]
    cs = cs_ref[p]

    def att_feat(x):
        t = jnp.dot(x, wa1_ref[...], preferred_element_type=jnp.float32)
        t = jnp.maximum(t + ba1_ref[...], 0.0)
        t = jnp.dot(t, wa2_ref[...], preferred_element_type=jnp.float32)
        return t + ba2_ref[...]

    rows = lax.broadcasted_iota(jnp.int32, (MAX_SET, 1), 0)
    qm = (rows < qs).astype(jnp.float32)
    cm = (rows < cs).astype(jnp.float32)
    mq = att_feat(q) * qm
    mc = att_feat(c) * cm
    la = lax.dot_general(mq, mc, (((1,), (1,)), ((), ())),
                         preferred_element_type=jnp.float32)
    pm = qm * cm.reshape(1, MAX_SET)
    masked = jnp.where(pm > 0.0, la * (1.0 / TEMP), -1e10)
    mx1 = jnp.max(masked, axis=1, keepdims=True)
    e1 = jnp.exp(masked - mx1)
    q_to_c = e1 / jnp.sum(e1, axis=1, keepdims=True)
    mx0 = jnp.max(masked, axis=0, keepdims=True)
    e0 = jnp.exp(masked - mx0)
    c_to_q = e0 / jnp.sum(e0, axis=0, keepdims=True)
    qd = q - jnp.dot(q_to_c, c, preferred_element_type=jnp.float32)
    q_score = -jnp.sum(jnp.sqrt(jnp.sum(qd * qd, axis=1) + 1e-12))
    cd = c - lax.dot_general(c_to_q, q, (((0,), (0,)), ((), ())),
                             preferred_element_type=jnp.float32)
    c_score = -jnp.sum(jnp.sqrt(jnp.sum(cd * cd, axis=1) + 1e-12))
    out_ref[...] = jnp.full((1, 1, D_NODE), jnp.maximum(q_score, c_score),
                            jnp.float32)


def _attention(h3, query_sizes, corpus_sizes, W_a1, b_a1, W_a2, b_a2):
    return pl.pallas_call(
        _attn_body,
        grid=(NUM_PAIRS,),
        in_specs=[
            pl.BlockSpec((1, NODES_PER_GRAPH, HALF), lambda p: (0, 2 * p, 0)),
            pl.BlockSpec((1, NODES_PER_GRAPH, HALF), lambda p: (1, 2 * p, 0)),
            pl.BlockSpec((1, NODES_PER_GRAPH, HALF), lambda p: (0, 2 * p + 1, 0)),
            pl.BlockSpec((1, NODES_PER_GRAPH, HALF), lambda p: (1, 2 * p + 1, 0)),
            pl.BlockSpec(memory_space=pltpu.MemorySpace.SMEM),
            pl.BlockSpec(memory_space=pltpu.MemorySpace.SMEM),
            pl.BlockSpec((D_NODE, D_ATT), lambda p: (0, 0)),
            pl.BlockSpec((1, D_ATT), lambda p: (0, 0)),
            pl.BlockSpec((D_ATT, D_ATT), lambda p: (0, 0)),
            pl.BlockSpec((1, D_ATT), lambda p: (0, 0)),
        ],
        out_specs=pl.BlockSpec((1, 1, D_NODE), lambda p: (p, 0, 0)),
        out_shape=jax.ShapeDtypeStruct((NUM_PAIRS, 1, D_NODE), jnp.float32),
    )(h3, h3, h3, h3, query_sizes, corpus_sizes, W_a1, b_a1, W_a2, b_a2)


def kernel(node_features, edge_features, from_idx, to_idx, query_sizes,
           corpus_sizes, W_ne, b_ne, W_ee, b_ee, W_msg, b_msg, W_upd, b_upd,
           W_a1, b_a1, W_a2, b_a2):
    from2d = from_idx.astype(jnp.int32).reshape(EDGE_ROWS, CHUNK)
    to2d = to_idx.astype(jnp.int32).reshape(EDGE_ROWS, CHUNK)

    M8 = jnp.zeros((8, D_EDGE), jnp.float32)
    M8 = M8.at[:D_EDGE_IN].set(W_ee)
    M8 = M8.at[4].set(b_ee)
    bm8 = jnp.zeros((8, D_NODE), jnp.float32)
    bm8 = bm8.at[4].set(b_msg)

    Wcat, Wtp, We8 = _prep_weights(W_msg, W_upd, M8, bm8)
    h3 = _encoder(node_features, W_ne, b_ne.reshape(1, D_NODE))
    e2 = _pass0(edge_features, to2d)

    zeros = jnp.zeros((ROWS_PER_TILE, HALF), jnp.float32)
    b_upd2 = b_upd.reshape(1, D_NODE)
    for _ in range(STEPS):
        a3 = _spmm(h3, from2d, to2d, zeros)
        h3 = _step(h3, a3, e2, Wcat, Wtp, We8, b_upd2)

    scores = _attention(h3, query_sizes.astype(jnp.int32),
                        corpus_sizes.astype(jnp.int32),
                        W_a1, b_a1.reshape(1, D_ATT), W_a2,
                        b_a2.reshape(1, D_ATT))
    return scores[:, 0, 0]


# speedup vs baseline: 7.3106x; 1.5674x over previous
import jax
import jax.numpy as jnp
from jax import lax
from jax.experimental import pallas as pl
from jax.experimental.pallas import tpu as pltpu
from jax.experimental.pallas import tpu_sc as plsc

N_NODES = 24576
N_EDGES = 393216
D_NODE = 128
D_EDGE_IN = 4
D_EDGE = 16
D_ATT = 64
NUM_PAIRS = 256
NODES_PER_GRAPH = 48
MAX_SET = 64
STEPS = 3
TEMP = 0.1

NC = 2
NS = 16
HALF = D_NODE // NC
EXP_ROWS = N_NODES // NS
CHUNK = 128
EDGE_ROWS = N_EDGES // CHUNK
ROWS_PER_TILE = EDGE_ROWS // NS
P0_ROWS = EDGE_ROWS // (NC * NS)
NBUF = 2
IDX_G = 48
OUTER = ROWS_PER_TILE // IDX_G

BLK = 512
N_BLKS = N_NODES // BLK


def _prep_body(wmsg_ref, wupd_ref, m8_ref, bm8_ref, wcat_ref, wtp_ref, we8_ref):
    wub = wupd_ref[D_NODE:, :]
    wcat_ref[:D_NODE, :] = wupd_ref[:D_NODE, :]
    wcat_ref[D_NODE:, :] = jnp.dot(wmsg_ref[:D_NODE, :], wub,
                                   preferred_element_type=jnp.float32)
    wtp_ref[...] = jnp.dot(wmsg_ref[D_NODE:2 * D_NODE, :], wub,
                           preferred_element_type=jnp.float32)
    tmp = jnp.dot(m8_ref[...], wmsg_ref[2 * D_NODE:, :],
                  preferred_element_type=jnp.float32) + bm8_ref[...]
    we8_ref[...] = jnp.dot(tmp, wub, preferred_element_type=jnp.float32)


def _prep_weights(W_msg, W_upd, M8, bm8):
    return pl.pallas_call(
        _prep_body,
        out_shape=(
            jax.ShapeDtypeStruct((2 * D_NODE, D_NODE), jnp.float32),
            jax.ShapeDtypeStruct((D_NODE, D_NODE), jnp.float32),
            jax.ShapeDtypeStruct((8, D_NODE), jnp.float32),
        ),
    )(W_msg, W_upd, M8, bm8)


def _enc_body(x_ref, w_ref, b_ref, out_ref):
    y = jnp.dot(x_ref[...], w_ref[...], preferred_element_type=jnp.float32)
    y = y + b_ref[...]
    out_ref[0] = y[:, :HALF]
    out_ref[1] = y[:, HALF:]


def _encoder(x, W_ne, b_ne):
    return pl.pallas_call(
        _enc_body,
        grid=(N_BLKS,),
        in_specs=[
            pl.BlockSpec((BLK, D_NODE), lambda i: (i, 0)),
            pl.BlockSpec((D_NODE, D_NODE), lambda i: (0, 0)),
            pl.BlockSpec((1, D_NODE), lambda i: (0, 0)),
        ],
        out_specs=pl.BlockSpec((NC, BLK, HALF), lambda i: (0, i, 0)),
        out_shape=jax.ShapeDtypeStruct((NC, N_NODES, HALF), jnp.float32),
    )(x, W_ne, b_ne)


def _pass0_body(e2d, to2d, zrows, zacc, out, tbuf, ev, rows8, acc8):
    c = lax.axis_index("c")
    s = lax.axis_index("s")
    pltpu.sync_copy(zacc, acc8.at[pl.ds(s * EXP_ROWS, EXP_ROWS)])
    pltpu.sync_copy(zrows, rows8)
    lane = jnp.arange(16, dtype=jnp.int32)
    ones = jnp.ones((16,), jnp.float32)
    col4 = jnp.full((16,), 4, jnp.int32)
    for k in range(8):
        plsc.store_scatter(rows8, [k * 16 + lane, col4], ones)
    base = c * (NS * P0_ROWS) + s * P0_ROWS
    pltpu.sync_copy(to2d.at[pl.ds(base, P0_ROWS)], tbuf)
    rowpat = lane >> 2
    colpat = lane & 3
    plsc.subcore_barrier()

    def chunk(r, carry):
        pltpu.sync_copy(e2d.at[pl.ds((base + r) * 4, 4)], ev)
        for j in range(4):
            for k in range(8):
                v = ev[j, pl.ds(k * 16, 16)]
                base_t = j * 32 + k * 4
                plsc.store_scatter(rows8, [base_t + rowpat, colpat], v)
        pltpu.sync_copy(rows8, acc8.at[tbuf.at[r]], add=True)
        return carry

    lax.fori_loop(0, P0_ROWS, chunk, 0)
    plsc.subcore_barrier()
    sl = pl.ds(s * EXP_ROWS, EXP_ROWS)
    pltpu.sync_copy(acc8.at[sl], out.at[c].at[sl])


def _pass0(edge_features, to2d):
    e2d = edge_features.reshape(N_EDGES * D_EDGE_IN // CHUNK, CHUNK)
    mesh = plsc.VectorSubcoreMesh(core_axis_name="c", subcore_axis_name="s",
                                  num_cores=NC, num_subcores=NS)
    zrows = jnp.zeros((CHUNK, 8), jnp.float32)
    zacc = jnp.zeros((EXP_ROWS, 8), jnp.float32)
    f = pl.kernel(
        _pass0_body,
        out_type=jax.ShapeDtypeStruct((NC, N_NODES, 8), jnp.float32),
        mesh=mesh,
        compiler_params=pltpu.CompilerParams(use_tc_tiling_on_sc=False,
                                             needs_layout_passes=False),
        scratch_types=[
            pltpu.VMEM((P0_ROWS, CHUNK), jnp.int32),
            pltpu.VMEM((4, CHUNK), jnp.float32),
            pltpu.VMEM((CHUNK, 8), jnp.float32),
            pltpu.VMEM_SHARED((N_NODES, 8), jnp.float32),
        ],
    )
    return f(e2d, to2d, zrows, zacc)


def _spmm_body(h3, from2d, to2d, zeros, out, fbuf, tbuf, rows, acc, sems):
    c = lax.axis_index("c")
    s = lax.axis_index("s")
    pltpu.sync_copy(zeros, acc.at[pl.ds(s * EXP_ROWS, EXP_ROWS)])
    plsc.subcore_barrier()

    def outer(o, carry):
        base = s * ROWS_PER_TILE + o * IDX_G
        pltpu.sync_copy(from2d.at[pl.ds(base, IDX_G)], fbuf)
        pltpu.sync_copy(to2d.at[pl.ds(base, IDX_G)], tbuf)
        for b in range(NBUF):
            pltpu.async_copy(h3.at[c].at[fbuf.at[b]], rows.at[b], sems.at[b])

        def chunk(r, carry2):
            b = lax.rem(r, NBUF)
            pltpu.make_async_copy(h3.at[c].at[fbuf.at[r]], rows.at[b],
                                  sems.at[b]).wait()
            pltpu.sync_copy(rows.at[b], acc.at[tbuf.at[r]], add=True)

            @pl.when(r + NBUF < IDX_G)
            def _():
                pltpu.async_copy(h3.at[c].at[fbuf.at[r + NBUF]], rows.at[b],
                                 sems.at[b])

            return carry2

        lax.fori_loop(0, IDX_G, chunk, 0)
        return carry

    lax.fori_loop(0, OUTER, outer, 0)
    plsc.subcore_barrier()
    sl = pl.ds(s * EXP_ROWS, EXP_ROWS)
    pltpu.sync_copy(acc.at[sl], out.at[c].at[sl])


def _spmm(h3, from2d, to2d, zeros):
    mesh = plsc.VectorSubcoreMesh(core_axis_name="c", subcore_axis_name="s",
                                  num_cores=NC, num_subcores=NS)
    f = pl.kernel(
        _spmm_body,
        out_type=jax.ShapeDtypeStruct((NC, N_NODES, HALF), jnp.float32),
        mesh=mesh,
        compiler_params=pltpu.CompilerParams(use_tc_tiling_on_sc=False),
        scratch_types=[
            pltpu.VMEM((IDX_G, CHUNK), jnp.int32),
            pltpu.VMEM((IDX_G, CHUNK), jnp.int32),
            pltpu.VMEM((NBUF, CHUNK, HALF), jnp.float32),
            pltpu.VMEM_SHARED((N_NODES, HALF), jnp.float32),
            pltpu.SemaphoreType.DMA((NBUF,)),
        ],
    )
    return f(h3, from2d, to2d, zeros)


def _step_body(h_ref, a_ref, e_ref, wcat_ref, wtp_ref, we8_ref, b_ref, out_ref):
    h = jnp.concatenate([h_ref[0], h_ref[1]], axis=1)
    a = jnp.concatenate([a_ref[0], a_ref[1]], axis=1)
    eraw = e_ref[0] + e_ref[1]
    cnt = eraw[:, 4:5]
    x = jnp.concatenate([h, a], axis=1)
    y = jnp.dot(x, wcat_ref[...], preferred_element_type=jnp.float32)
    y = y + jnp.dot(cnt * h, wtp_ref[...], preferred_element_type=jnp.float32)
    y = y + jnp.dot(eraw, we8_ref[...], preferred_element_type=jnp.float32)
    y = y + b_ref[...]
    out_ref[0] = y[:, :HALF]
    out_ref[1] = y[:, HALF:]


def _step(h3, a3, e2, Wcat, Wtp, We8, b_upd):
    return pl.pallas_call(
        _step_body,
        grid=(N_BLKS,),
        in_specs=[
            pl.BlockSpec((NC, BLK, HALF), lambda i: (0, i, 0)),
            pl.BlockSpec((NC, BLK, HALF), lambda i: (0, i, 0)),
            pl.BlockSpec((NC, BLK, 8), lambda i: (0, i, 0)),
            pl.BlockSpec((2 * D_NODE, D_NODE), lambda i: (0, 0)),
            pl.BlockSpec((D_NODE, D_NODE), lambda i: (0, 0)),
            pl.BlockSpec((8, D_NODE), lambda i: (0, 0)),
            pl.BlockSpec((1, D_NODE), lambda i: (0, 0)),
        ],
        out_specs=pl.BlockSpec((NC, BLK, HALF), lambda i: (0, i, 0)),
        out_shape=jax.ShapeDtypeStruct((NC, N_NODES, HALF), jnp.float32),
    )(h3, a3, e2, Wcat, Wtp, We8, b_upd)


def _attn_body(qlo_ref, qhi_ref, clo_ref, chi_ref, qs_ref, cs_ref,
               wa1_ref, ba1_ref, wa2_ref, ba2_ref, out_ref):
    p = pl.program_id(0)
    zpad = jnp.zeros((MAX_SET - NODES_PER_GRAPH, D_NODE), jnp.float32)
    q = jnp.concatenate(
        [jnp.concatenate([qlo_ref[0], qhi_ref[0]], axis=1), zpad], axis=0)
    c = jnp.concatenate(
        [jnp.concatenate([clo_ref[0], chi_ref[0]], axis=1), zpad], axis=0)
    qs = qs_ref[p]
    cs = cs_ref[p]

    def att_feat(x):
        t = jnp.dot(x, wa1_ref[...], preferred_element_type=jnp.float32)
        t = jnp.maximum(t + ba1_ref[...], 0.0)
        t = jnp.dot(t, wa2_ref[...], preferred_element_type=jnp.float32)
        return t + ba2_ref[...]

    rows = lax.broadcasted_iota(jnp.int32, (MAX_SET, 1), 0)
    qm = (rows < qs).astype(jnp.float32)
    cm = (rows < cs).astype(jnp.float32)
    mq = att_feat(q) * qm
    mc = att_feat(c) * cm
    la = lax.dot_general(mq, mc, (((1,), (1,)), ((), ())),
                         preferred_element_type=jnp.float32)
    pm = qm * cm.reshape(1, MAX_SET)
    masked = jnp.where(pm > 0.0, la * (1.0 / TEMP), -1e10)
    mx1 = jnp.max(masked, axis=1, keepdims=True)
    e1 = jnp.exp(masked - mx1)
    q_to_c = e1 / jnp.sum(e1, axis=1, keepdims=True)
    mx0 = jnp.max(masked, axis=0, keepdims=True)
    e0 = jnp.exp(masked - mx0)
    c_to_q = e0 / jnp.sum(e0, axis=0, keepdims=True)
    qd = q - jnp.dot(q_to_c, c, preferred_element_type=jnp.float32)
    q_score = -jnp.sum(jnp.sqrt(jnp.sum(qd * qd, axis=1) + 1e-12))
    cd = c - lax.dot_general(c_to_q, q, (((0,), (0,)), ((), ())),
                             preferred_element_type=jnp.float32)
    c_score = -jnp.sum(jnp.sqrt(jnp.sum(cd * cd, axis=1) + 1e-12))
    out_ref[...] = jnp.full((1, 1, D_NODE), jnp.maximum(q_score, c_score),
                            jnp.float32)


def _attention(h3, query_sizes, corpus_sizes, W_a1, b_a1, W_a2, b_a2):
    return pl.pallas_call(
        _attn_body,
        grid=(NUM_PAIRS,),
        in_specs=[
            pl.BlockSpec((1, NODES_PER_GRAPH, HALF), lambda p: (0, 2 * p, 0)),
            pl.BlockSpec((1, NODES_PER_GRAPH, HALF), lambda p: (1, 2 * p, 0)),
            pl.BlockSpec((1, NODES_PER_GRAPH, HALF), lambda p: (0, 2 * p + 1, 0)),
            pl.BlockSpec((1, NODES_PER_GRAPH, HALF), lambda p: (1, 2 * p + 1, 0)),
            pl.BlockSpec(memory_space=pltpu.MemorySpace.SMEM),
            pl.BlockSpec(memory_space=pltpu.MemorySpace.SMEM),
            pl.BlockSpec((D_NODE, D_ATT), lambda p: (0, 0)),
            pl.BlockSpec((1, D_ATT), lambda p: (0, 0)),
            pl.BlockSpec((D_ATT, D_ATT), lambda p: (0, 0)),
            pl.BlockSpec((1, D_ATT), lambda p: (0, 0)),
        ],
        out_specs=pl.BlockSpec((1, 1, D_NODE), lambda p: (p, 0, 0)),
        out_shape=jax.ShapeDtypeStruct((NUM_PAIRS, 1, D_NODE), jnp.float32),
    )(h3, h3, h3, h3, query_sizes, corpus_sizes, W_a1, b_a1, W_a2, b_a2)


def kernel(node_features, edge_features, from_idx, to_idx, query_sizes,
           corpus_sizes, W_ne, b_ne, W_ee, b_ee, W_msg, b_msg, W_upd, b_upd,
           W_a1, b_a1, W_a2, b_a2):
    from2d = from_idx.astype(jnp.int32).reshape(EDGE_ROWS, CHUNK)
    to2d = to_idx.astype(jnp.int32).reshape(EDGE_ROWS, CHUNK)

    M8 = jnp.zeros((8, D_EDGE), jnp.float32)
    M8 = M8.at[:D_EDGE_IN].set(W_ee)
    M8 = M8.at[4].set(b_ee)
    bm8 = jnp.zeros((8, D_NODE), jnp.float32)
    bm8 = bm8.at[4].set(b_msg)

    Wcat, Wtp, We8 = _prep_weights(W_msg, W_upd, M8, bm8)
    h3 = _encoder(node_features, W_ne, b_ne.reshape(1, D_NODE))
    e2 = _pass0(edge_features, to2d)

    zeros = jnp.zeros((EXP_ROWS, HALF), jnp.float32)
    b_upd2 = b_upd.reshape(1, D_NODE)
    for _ in range(STEPS):
        a3 = _spmm(h3, from2d, to2d, zeros)
        h3 = _step(h3, a3, e2, Wcat, Wtp, We8, b_upd2)

    scores = _attention(h3, query_sizes.astype(jnp.int32),
                        corpus_sizes.astype(jnp.int32),
                        W_a1, b_a1.reshape(1, D_ATT), W_a2,
                        b_a2.reshape(1, D_ATT))
    return scores[:, 0, 0]


# speedup vs baseline: 13.1349x; 1.7967x over previous
import jax
import jax.numpy as jnp
from jax import lax
from jax.experimental import pallas as pl
from jax.experimental.pallas import tpu as pltpu
from jax.experimental.pallas import tpu_sc as plsc

N_NODES = 24576
N_EDGES = 393216
D_NODE = 128
D_EDGE_IN = 4
D_EDGE = 16
D_ATT = 64
NUM_PAIRS = 256
NODES_PER_GRAPH = 48
MAX_SET = 64
STEPS = 3
TEMP = 0.1

NC = 2
NS = 16
HALF = D_NODE // NC
EXP_ROWS = N_NODES // NS
CHUNK = 128
EDGE_ROWS = N_EDGES // CHUNK
ROWS_PER_TILE = EDGE_ROWS // NS
P0_ROWS = EDGE_ROWS // (NC * NS)
NBUF = 3
IDX_G = 24
OUTER = ROWS_PER_TILE // IDX_G

BLK = 512
N_BLKS = N_NODES // BLK


def _prep_body(wmsg_ref, wupd_ref, m8_ref, bm8_ref, wcat_ref, wtp_ref, we8_ref):
    wub = wupd_ref[D_NODE:, :]
    wcat_ref[:D_NODE, :] = wupd_ref[:D_NODE, :]
    wcat_ref[D_NODE:, :] = jnp.dot(wmsg_ref[:D_NODE, :], wub,
                                   preferred_element_type=jnp.float32)
    wtp_ref[...] = jnp.dot(wmsg_ref[D_NODE:2 * D_NODE, :], wub,
                           preferred_element_type=jnp.float32)
    tmp = jnp.dot(m8_ref[...], wmsg_ref[2 * D_NODE:, :],
                  preferred_element_type=jnp.float32) + bm8_ref[...]
    we8_ref[...] = jnp.dot(tmp, wub, preferred_element_type=jnp.float32)


def _prep_weights(W_msg, W_upd, M8, bm8):
    return pl.pallas_call(
        _prep_body,
        out_shape=(
            jax.ShapeDtypeStruct((2 * D_NODE, D_NODE), jnp.float32),
            jax.ShapeDtypeStruct((D_NODE, D_NODE), jnp.float32),
            jax.ShapeDtypeStruct((8, D_NODE), jnp.float32),
        ),
    )(W_msg, W_upd, M8, bm8)


NP = N_NODES // 2
BLK2 = 256


def _expand128(W):
    M = jnp.zeros((256, 256), jnp.float32)
    for a in range(2):
        for ib in range(2):
            for jb in range(2):
                M = M.at[ib * 128 + a * 64:ib * 128 + a * 64 + 64,
                         jb * 128 + a * 64:jb * 128 + a * 64 + 64].set(
                             W[ib * 64:(ib + 1) * 64, jb * 64:(jb + 1) * 64])
    return M


def _expand128_seq(W):
    M = jnp.zeros((256, 256), jnp.float32)
    for a in range(2):
        for jb in range(2):
            M = M.at[a * 128:(a + 1) * 128,
                     jb * 128 + a * 64:jb * 128 + a * 64 + 64].set(
                         W[:, jb * 64:(jb + 1) * 64])
    return M


def _tile_bias(b):
    return jnp.concatenate([b[:64], b[:64], b[64:], b[64:]]).reshape(1, 256)


def _enc_body(x_ref, w_ref, b_ref, out_ref):
    y = jnp.dot(x_ref[...], w_ref[...], preferred_element_type=jnp.float32)
    y = y + b_ref[...]
    out_ref[0] = y[:, :D_NODE]
    out_ref[1] = y[:, D_NODE:]


def _encoder(xp, Mne, bneI):
    return pl.pallas_call(
        _enc_body,
        grid=(NP // BLK2,),
        in_specs=[
            pl.BlockSpec((BLK2, 2 * D_NODE), lambda i: (i, 0)),
            pl.BlockSpec((2 * D_NODE, 2 * D_NODE), lambda i: (0, 0)),
            pl.BlockSpec((1, 2 * D_NODE), lambda i: (0, 0)),
        ],
        out_specs=pl.BlockSpec((NC, BLK2, D_NODE), lambda i: (0, i, 0)),
        out_shape=jax.ShapeDtypeStruct((NC, NP, D_NODE), jnp.float32),
    )(xp, Mne, bneI)


def _esum_body(e_ref, out_ref):
    out_ref[...] = e_ref[0] + e_ref[1]


def _esum(e2):
    return pl.pallas_call(
        _esum_body,
        grid=(N_BLKS,),
        in_specs=[pl.BlockSpec((NC, BLK, 8), lambda i: (0, i, 0))],
        out_specs=pl.BlockSpec((BLK, 8), lambda i: (i, 0)),
        out_shape=jax.ShapeDtypeStruct((N_NODES, 8), jnp.float32),
    )(e2)


def _pass0_body(e2d, to2d, zrows, zacc, out, tbuf, ev, rows8, acc8):
    c = lax.axis_index("c")
    s = lax.axis_index("s")
    pltpu.sync_copy(zacc, acc8.at[pl.ds(s * EXP_ROWS, EXP_ROWS)])
    pltpu.sync_copy(zrows, rows8)
    lane = jnp.arange(16, dtype=jnp.int32)
    ones = jnp.ones((16,), jnp.float32)
    col4 = jnp.full((16,), 4, jnp.int32)
    for k in range(8):
        plsc.store_scatter(rows8, [k * 16 + lane, col4], ones)
    base = c * (NS * P0_ROWS) + s * P0_ROWS
    pltpu.sync_copy(to2d.at[pl.ds(base, P0_ROWS)], tbuf)
    pltpu.sync_copy(e2d.at[pl.ds(base * 4, P0_ROWS * 4)], ev)
    plsc.subcore_barrier()

    def chunk(r, carry):
        for j in range(4):
            colj = jnp.full((16,), j, jnp.int32)
            for g in range(8):
                v = ev[r * 4 + j, pl.ds(g * 16, 16)]
                plsc.store_scatter(rows8, [g * 16 + lane, colj], v)
        pltpu.sync_copy(rows8, acc8.at[tbuf.at[r]], add=True)
        return carry

    lax.fori_loop(0, P0_ROWS, chunk, 0)
    plsc.subcore_barrier()
    sl = pl.ds(s * EXP_ROWS, EXP_ROWS)
    pltpu.sync_copy(acc8.at[sl], out.at[c].at[sl])


def _pass0(edge_features, to2d):
    e2d = (jnp.swapaxes(edge_features, 0, 1)
           .reshape(D_EDGE_IN, EDGE_ROWS, CHUNK)
           .transpose(1, 0, 2)
           .reshape(EDGE_ROWS * D_EDGE_IN, CHUNK))
    mesh = plsc.VectorSubcoreMesh(core_axis_name="c", subcore_axis_name="s",
                                  num_cores=NC, num_subcores=NS)
    zrows = jnp.zeros((CHUNK, 8), jnp.float32)
    zacc = jnp.zeros((EXP_ROWS, 8), jnp.float32)
    f = pl.kernel(
        _pass0_body,
        out_type=jax.ShapeDtypeStruct((NC, N_NODES, 8), jnp.float32),
        mesh=mesh,
        compiler_params=pltpu.CompilerParams(use_tc_tiling_on_sc=False,
                                             needs_layout_passes=False),
        scratch_types=[
            pltpu.VMEM((P0_ROWS, CHUNK), jnp.int32),
            pltpu.VMEM((P0_ROWS * 4, CHUNK), jnp.float32),
            pltpu.VMEM((CHUNK, 8), jnp.float32),
            pltpu.VMEM_SHARED((N_NODES, 8), jnp.float32),
        ],
    )
    return f(e2d, to2d, zrows, zacc)


def _spmm_body(h3, from2d, to2d, zeros, out, fbuf, tbuf, rows, acc, sems):
    c = lax.axis_index("c")
    s = lax.axis_index("s")
    pltpu.sync_copy(zeros, acc.at[pl.ds(s * EXP_ROWS, EXP_ROWS)])
    plsc.subcore_barrier()

    def outer(o, carry):
        base = s * ROWS_PER_TILE + o * IDX_G
        pltpu.sync_copy(from2d.at[pl.ds(base, IDX_G)], fbuf)
        pltpu.sync_copy(to2d.at[pl.ds(base, IDX_G)], tbuf)
        for b in range(NBUF):
            pltpu.async_copy(h3.at[c].at[fbuf.at[b]], rows.at[b], sems.at[b])

        def chunk(r, carry2):
            b = lax.rem(r, NBUF)
            pltpu.make_async_copy(h3.at[c].at[fbuf.at[r]], rows.at[b],
                                  sems.at[b]).wait()
            pltpu.sync_copy(rows.at[b], acc.at[tbuf.at[r]], add=True)

            @pl.when(r + NBUF < IDX_G)
            def _():
                pltpu.async_copy(h3.at[c].at[fbuf.at[r + NBUF]], rows.at[b],
                                 sems.at[b])

            return carry2

        lax.fori_loop(0, IDX_G, chunk, 0)
        return carry

    lax.fori_loop(0, OUTER, outer, 0)
    plsc.subcore_barrier()
    sl = pl.ds(s * EXP_ROWS, EXP_ROWS)
    pltpu.sync_copy(acc.at[sl], out.at[c].at[sl])


def _spmm(h3, from2d, to2d, zeros):
    mesh = plsc.VectorSubcoreMesh(core_axis_name="c", subcore_axis_name="s",
                                  num_cores=NC, num_subcores=NS)
    f = pl.kernel(
        _spmm_body,
        out_type=jax.ShapeDtypeStruct((NC, N_NODES, HALF), jnp.float32),
        mesh=mesh,
        compiler_params=pltpu.CompilerParams(use_tc_tiling_on_sc=False),
        scratch_types=[
            pltpu.VMEM((IDX_G, CHUNK), jnp.int32),
            pltpu.VMEM((IDX_G, CHUNK), jnp.int32),
            pltpu.VMEM((NBUF, CHUNK, HALF), jnp.float32),
            pltpu.VMEM_SHARED((N_NODES, HALF), jnp.float32),
            pltpu.SemaphoreType.DMA((NBUF,)),
        ],
    )
    return f(h3, from2d, to2d, zeros)


def _step_body(h_ref, a_ref, e_ref, mcat_ref, u_ref, we16_ref, b_ref, out_ref):
    xh = jnp.concatenate([h_ref[0], h_ref[1]], axis=1)
    x = jnp.concatenate([xh, a_ref[0], a_ref[1]], axis=1)
    e16 = e_ref[...]
    cp = e16[:, 4:5]
    cq = e16[:, 12:13]
    ones64 = jnp.ones((1, 64), jnp.float32)
    cnt_i = jnp.concatenate([cp * ones64, cq * ones64,
                             cp * ones64, cq * ones64], axis=1)
    y = jnp.dot(x, mcat_ref[...], preferred_element_type=jnp.float32)
    y = y + jnp.dot(cnt_i * xh, u_ref[...], preferred_element_type=jnp.float32)
    y = y + jnp.dot(e16, we16_ref[...], preferred_element_type=jnp.float32)
    y = y + b_ref[...]
    out_ref[0] = y[:, :D_NODE]
    out_ref[1] = y[:, D_NODE:]


def _step(P, aP, e16, Mcat, U, We16, bI):
    return pl.pallas_call(
        _step_body,
        grid=(NP // BLK2,),
        in_specs=[
            pl.BlockSpec((NC, BLK2, D_NODE), lambda i: (0, i, 0)),
            pl.BlockSpec((NC, BLK2, D_NODE), lambda i: (0, i, 0)),
            pl.BlockSpec((BLK2, 16), lambda i: (i, 0)),
            pl.BlockSpec((4 * D_NODE, 2 * D_NODE), lambda i: (0, 0)),
            pl.BlockSpec((2 * D_NODE, 2 * D_NODE), lambda i: (0, 0)),
            pl.BlockSpec((16, 2 * D_NODE), lambda i: (0, 0)),
            pl.BlockSpec((1, 2 * D_NODE), lambda i: (0, 0)),
        ],
        out_specs=pl.BlockSpec((NC, BLK2, D_NODE), lambda i: (0, i, 0)),
        out_shape=jax.ShapeDtypeStruct((NC, NP, D_NODE), jnp.float32),
    )(P, aP, e16, Mcat, U, We16, bI)


PAIRS_PER_PROG = 8


def _attn_body(x_ref, qs_ref, cs_ref,
               wa1_ref, ba1_ref, wa2_ref, ba2_ref, out_ref):
    p0 = pl.program_id(0) * PAIRS_PER_PROG
    zpad = jnp.zeros((MAX_SET - NODES_PER_GRAPH, D_NODE), jnp.float32)
    rows = lax.broadcasted_iota(jnp.int32, (MAX_SET, 1), 0)

    def att_feat(x):
        t = jnp.dot(x, wa1_ref[...], preferred_element_type=jnp.float32)
        t = jnp.maximum(t + ba1_ref[...], 0.0)
        t = jnp.dot(t, wa2_ref[...], preferred_element_type=jnp.float32)
        return t + ba2_ref[...]

    for g in range(PAIRS_PER_PROG):
        q = jnp.concatenate(
            [x_ref[g * 96:g * 96 + NODES_PER_GRAPH, :], zpad], axis=0)
        c = jnp.concatenate(
            [x_ref[g * 96 + NODES_PER_GRAPH:(g + 1) * 96, :], zpad], axis=0)
        qs = qs_ref[p0 + g]
        cs = cs_ref[p0 + g]
        qm = (rows < qs).astype(jnp.float32)
        cm = (rows < cs).astype(jnp.float32)
        mq = att_feat(q) * qm
        mc = att_feat(c) * cm
        la = lax.dot_general(mq, mc, (((1,), (1,)), ((), ())),
                             preferred_element_type=jnp.float32)
        pm = qm * cm.reshape(1, MAX_SET)
        masked = jnp.where(pm > 0.0, la * (1.0 / TEMP), -1e10)
        mx1 = jnp.max(masked, axis=1, keepdims=True)
        e1 = jnp.exp(masked - mx1)
        q_to_c = e1 / jnp.sum(e1, axis=1, keepdims=True)
        mx0 = jnp.max(masked, axis=0, keepdims=True)
        e0 = jnp.exp(masked - mx0)
        c_to_q = e0 / jnp.sum(e0, axis=0, keepdims=True)
        qd = q - jnp.dot(q_to_c, c, preferred_element_type=jnp.float32)
        q_score = -jnp.sum(jnp.sqrt(jnp.sum(qd * qd, axis=1) + 1e-12))
        cd = c - lax.dot_general(c_to_q, q, (((0,), (0,)), ((), ())),
                                 preferred_element_type=jnp.float32)
        c_score = -jnp.sum(jnp.sqrt(jnp.sum(cd * cd, axis=1) + 1e-12))
        out_ref[g, 0, :] = jnp.full((D_NODE,), jnp.maximum(q_score, c_score),
                                    jnp.float32)


def _attention(hF, query_sizes, corpus_sizes, W_a1, b_a1, W_a2, b_a2):
    return pl.pallas_call(
        _attn_body,
        grid=(NUM_PAIRS // PAIRS_PER_PROG,),
        in_specs=[
            pl.BlockSpec((PAIRS_PER_PROG * 96, D_NODE), lambda i: (i, 0)),
            pl.BlockSpec(memory_space=pltpu.MemorySpace.SMEM),
            pl.BlockSpec(memory_space=pltpu.MemorySpace.SMEM),
            pl.BlockSpec((D_NODE, D_ATT), lambda i: (0, 0)),
            pl.BlockSpec((1, D_ATT), lambda i: (0, 0)),
            pl.BlockSpec((D_ATT, D_ATT), lambda i: (0, 0)),
            pl.BlockSpec((1, D_ATT), lambda i: (0, 0)),
        ],
        out_specs=pl.BlockSpec((PAIRS_PER_PROG, 1, D_NODE), lambda i: (i, 0, 0)),
        out_shape=jax.ShapeDtypeStruct((NUM_PAIRS, 1, D_NODE), jnp.float32),
    )(hF, query_sizes, corpus_sizes, W_a1, b_a1, W_a2, b_a2)


def kernel(node_features, edge_features, from_idx, to_idx, query_sizes,
           corpus_sizes, W_ne, b_ne, W_ee, b_ee, W_msg, b_msg, W_upd, b_upd,
           W_a1, b_a1, W_a2, b_a2):
    from2d = from_idx.astype(jnp.int32).reshape(EDGE_ROWS, CHUNK)
    to2d = to_idx.astype(jnp.int32).reshape(EDGE_ROWS, CHUNK)

    M8 = jnp.zeros((8, D_EDGE), jnp.float32)
    M8 = M8.at[:D_EDGE_IN].set(W_ee)
    M8 = M8.at[4].set(b_ee)
    bm8 = jnp.zeros((8, D_NODE), jnp.float32)
    bm8 = bm8.at[4].set(b_msg)

    Wcat, Wtp, We8 = _prep_weights(W_msg, W_upd, M8, bm8)

    Mne = _expand128_seq(W_ne)
    bneI = _tile_bias(b_ne)
    Mcat = jnp.concatenate([_expand128(Wcat[:D_NODE]),
                            _expand128(Wcat[D_NODE:])], axis=0)
    U = _expand128(Wtp)
    We16 = jnp.zeros((16, 256), jnp.float32)
    for a in range(2):
        for jb in range(2):
            We16 = We16.at[a * 8:(a + 1) * 8,
                           jb * 128 + a * 64:jb * 128 + a * 64 + 64].set(
                               We8[:, jb * 64:(jb + 1) * 64])
    bI = _tile_bias(b_upd)

    xp = node_features.reshape(NP, 2 * D_NODE)
    P = _encoder(xp, Mne, bneI)
    e2 = _pass0(edge_features, to2d)
    e16 = _esum(e2).reshape(NP, 16)

    zeros = jnp.zeros((EXP_ROWS, HALF), jnp.float32)
    for _ in range(STEPS):
        a3 = _spmm(P.reshape(NC, N_NODES, HALF), from2d, to2d, zeros)
        P = _step(P, a3.reshape(NC, NP, D_NODE), e16, Mcat, U, We16, bI)

    hF = (P.reshape(2, NP, 2, HALF).transpose(1, 2, 0, 3)
          .reshape(N_NODES, D_NODE))
    scores = _attention(hF, query_sizes.astype(jnp.int32),
                        corpus_sizes.astype(jnp.int32),
                        W_a1, b_a1.reshape(1, D_ATT), W_a2,
                        b_a2.reshape(1, D_ATT))
    return scores[:, 0, 0]
